# baseline (speedup 1.0000x reference)
import functools
import jax
import jax.numpy as jnp
from jax import lax
from jax.experimental import pallas as pl
from jax.experimental.pallas import tpu as pltpu
from jax.experimental.pallas import tpu_sc as plsc

_EPS = 1e-8
_VPB = 4


def _sc_passthrough(vhb):
    nv = vhb.shape[0]
    mesh = plsc.VectorSubcoreMesh(core_axis_name="c", subcore_axis_name="s",
                                  num_cores=2, num_subcores=16)

    @functools.partial(
        pl.kernel,
        out_type=jax.ShapeDtypeStruct((nv,), jnp.float32),
        mesh=mesh,
        scratch_types=[pltpu.VMEM((nv,), jnp.float32)],
    )
    def k(vhb_hbm, out_hbm, v_v):
        wid = lax.axis_index("s") * 2 + lax.axis_index("c")

        @pl.when(wid == 0)
        def _():
            pltpu.sync_copy(vhb_hbm, v_v)
            pltpu.sync_copy(v_v, out_hbm)

    return k(vhb)


def _tc_body(data_ref, mus_ref, ls_ref, vlow_ref, vhigh_ref, vhb_ref,
             out_ref):
    nrows, b = out_ref.shape
    npv = nrows // _VPB
    v0 = pl.program_id(0) * _VPB

    for h in range(_VPB):
        low = vlow_ref[v0 + h, 0]
        high = vhigh_ref[v0 + h, 0]
        hb = vhb_ref[v0 + h, 0]

        sd_row = (data_ref[0, h, :].reshape(1, b) - low) * (1.0 / (high - low))
        low_m = sd_row < 0.01
        high_m = sd_row > 0.99
        ar = jnp.where(low_m, 0.0, jnp.where(high_m, -0.5, 0.5 * (1.0 + _EPS)))
        al = jnp.where(low_m, 0.5, jnp.where(high_m, 0.0, -0.5))
        beta = jnp.where(low_m, 0.5, jnp.where(high_m, 0.5 + _EPS, 0.5 * _EPS))

        mu_row = mus_ref[0, h, :].reshape(1, npv)
        ls_row = jnp.maximum(ls_ref[0, h, :].reshape(1, npv), -5.0)
        isch_row = jnp.exp(-ls_row) * 0.5
        muh_row = (mu_row + hb) * isch_row
        dh_row = (2.0 * hb) * isch_row

        isch_c = isch_row.reshape(npv, 1)
        muh_c = muh_row.reshape(npv, 1)
        dh_c = dh_row.reshape(npv, 1)

        argl = sd_row * isch_c - muh_c
        tl = jnp.tanh(argl)
        tr = jnp.tanh(argl + dh_c)
        numer = ar * tr + al * tl + beta
        out_ref[h * npv:(h + 1) * npv, :] = jnp.log(numer)


def kernel(data, node_mars, mus, log_scales, vids, d2vids, vrangeslow,
           vrangeshigh, vhbinsizes):
    nv, b = data.shape
    nn = mus.shape[0]
    npv = nn // nv
    ng = nv // _VPB
    vhbinsizes = _sc_passthrough(vhbinsizes.reshape(nv)).reshape(nv, 1)
    return pl.pallas_call(
        _tc_body,
        grid=(ng,),
        in_specs=[
            pl.BlockSpec((1, _VPB, b), lambda v: (v, 0, 0)),
            pl.BlockSpec((1, _VPB, npv), lambda v: (v, 0, 0)),
            pl.BlockSpec((1, _VPB, npv), lambda v: (v, 0, 0)),
            pl.BlockSpec(memory_space=pltpu.SMEM),
            pl.BlockSpec(memory_space=pltpu.SMEM),
            pl.BlockSpec(memory_space=pltpu.SMEM),
        ],
        out_specs=pl.BlockSpec((_VPB * npv, b), lambda v: (v, 0)),
        out_shape=jax.ShapeDtypeStruct((nn, b), jnp.float32),
    )(data.reshape(ng, _VPB, b), mus.reshape(ng, _VPB, npv),
      log_scales.reshape(ng, _VPB, npv), vrangeslow, vrangeshigh, vhbinsizes)


# speedup vs baseline: 1.2338x; 1.2338x over previous
import jax
import jax.numpy as jnp
from jax.experimental import pallas as pl
from jax.experimental.pallas import tpu as pltpu

_EPS = 1e-8
_VPB = 8


def _tc_body(data_ref, mus_ref, ls_ref, vlow_ref, vhigh_ref, vhb_ref,
             out_ref):
    nrows, b = out_ref.shape
    npv = nrows // _VPB
    v0 = pl.program_id(0) * _VPB

    for h in range(_VPB):
        low = vlow_ref[v0 + h, 0]
        high = vhigh_ref[v0 + h, 0]
        hb = vhb_ref[v0 + h, 0]

        sd_row = (data_ref[0, h, :].reshape(1, b) - low) * (1.0 / (high - low))
        low_m = sd_row < 0.01
        high_m = sd_row > 0.99
        ar = jnp.where(low_m, 0.0, jnp.where(high_m, -0.5, 0.5 * (1.0 + _EPS)))
        al = jnp.where(low_m, 0.5, jnp.where(high_m, 0.0, -0.5))
        beta = jnp.where(low_m, 0.5, jnp.where(high_m, 0.5 + _EPS, 0.5 * _EPS))

        mu_row = mus_ref[0, h, :].reshape(1, npv)
        ls_row = jnp.maximum(ls_ref[0, h, :].reshape(1, npv), -5.0)
        isch_row = jnp.exp(-ls_row) * 0.5
        muh_row = (mu_row + hb) * isch_row
        dh_row = (2.0 * hb) * isch_row

        isch_c = isch_row.reshape(npv, 1)
        muh_c = muh_row.reshape(npv, 1)
        dh_c = dh_row.reshape(npv, 1)

        argl = sd_row * isch_c - muh_c
        tl = jnp.tanh(argl)
        tr = jnp.tanh(argl + dh_c)
        numer = ar * tr + al * tl + beta
        out_ref[h * npv:(h + 1) * npv, :] = jnp.log(numer)


def kernel(data, node_mars, mus, log_scales, vids, d2vids, vrangeslow,
           vrangeshigh, vhbinsizes):
    nv, b = data.shape
    nn = mus.shape[0]
    npv = nn // nv
    ng = nv // _VPB
    return pl.pallas_call(
        _tc_body,
        grid=(ng,),
        in_specs=[
            pl.BlockSpec((1, _VPB, b), lambda v: (v, 0, 0)),
            pl.BlockSpec((1, _VPB, npv), lambda v: (v, 0, 0)),
            pl.BlockSpec((1, _VPB, npv), lambda v: (v, 0, 0)),
            pl.BlockSpec(memory_space=pltpu.SMEM),
            pl.BlockSpec(memory_space=pltpu.SMEM),
            pl.BlockSpec(memory_space=pltpu.SMEM),
        ],
        out_specs=pl.BlockSpec((_VPB * npv, b), lambda v: (v, 0)),
        out_shape=jax.ShapeDtypeStruct((nn, b), jnp.float32),
    )(data.reshape(ng, _VPB, b), mus.reshape(ng, _VPB, npv),
      log_scales.reshape(ng, _VPB, npv), vrangeslow, vrangeshigh, vhbinsizes)
